# baseline (speedup 1.0000x reference)
import functools

import jax
import jax.numpy as jnp
from jax import lax
from jax.experimental import pallas as pl
from jax.experimental.pallas import tpu as pltpu
from jax.experimental.pallas import tpu_sc as plsc

D_MODEL = 2048
N_EXP = 64
TOPK = 8
N_TOK = 8192
BLK = 1024

_INFO = plsc.get_sparse_core_info()
_NC = _INFO.num_cores
_NS = _INFO.num_subcores
NW = _NC * _NS
ROWS = N_TOK // NW


def _dense_body(x_ref, w_ref, logits_ref, scores_ref):
    x = x_ref[...]
    w = w_ref[...]
    logits = jax.lax.dot_general(
        x, w, (((1,), (1,)), ((), ())), preferred_element_type=jnp.float32
    )
    logits_ref[...] = logits
    m = jnp.max(logits, axis=-1, keepdims=True)
    e = jnp.exp(logits - m)
    scores_ref[...] = e / jnp.sum(e, axis=-1, keepdims=True)


def _dense(x, W):
    return pl.pallas_call(
        _dense_body,
        grid=(N_TOK // BLK,),
        in_specs=[
            pl.BlockSpec((BLK, D_MODEL), lambda i: (i, 0)),
            pl.BlockSpec((N_EXP, D_MODEL), lambda i: (0, 0)),
        ],
        out_specs=[
            pl.BlockSpec((BLK, N_EXP), lambda i: (i, 0)),
            pl.BlockSpec((BLK, N_EXP), lambda i: (i, 0)),
        ],
        out_shape=[
            jax.ShapeDtypeStruct((N_TOK, N_EXP), jnp.float32),
            jax.ShapeDtypeStruct((N_TOK, N_EXP), jnp.float32),
        ],
    )(x, W)


def _sc_body(scores_hbm, ew_hbm, ei_hbm, hist_hbm, blk_v, ew_v, ei_v, hist_v):
    wid = lax.axis_index("s") * _NC + lax.axis_index("c")
    base = wid * ROWS

    pltpu.sync_copy(scores_hbm.at[pl.ds(base * N_EXP, ROWS * N_EXP)], blk_v)

    iota = lax.iota(jnp.int32, 16)
    lane_mask = iota < TOPK
    ones = jnp.ones((16,), jnp.int32)
    for j in range(4):
        hist_v[pl.ds(16 * j, 16)] = jnp.zeros((16,), jnp.int32)

    def merge(ka, va, kb, vb):
        kr = lax.rev(kb, (0,))
        vr = lax.rev(vb, (0,))
        sel = ka >= kr
        return plsc.sort_key_val(
            jnp.where(sel, ka, kr), jnp.where(sel, va, vr), descending=True
        )

    @plsc.parallel_loop(0, ROWS, 1, unroll=8)
    def _row_loop(r):
        off = r * N_EXP
        ks, vs = [], []
        for j in range(4):
            k, v = plsc.sort_key_val(
                blk_v[pl.ds(off + 16 * j, 16)], iota + 16 * j, descending=True
            )
            ks.append(k)
            vs.append(v)
        k01, v01 = merge(ks[0], vs[0], ks[1], vs[1])
        k23, v23 = merge(ks[2], vs[2], ks[3], vs[3])
        kf, vf = merge(k01, v01, k23, v23)
        ew_v[pl.ds(r * 16, 16)] = kf
        ei_v[pl.ds(r * 16, 16)] = vf
        plsc.addupdate_scatter(hist_v, [vf], ones, mask=lane_mask)

    pltpu.sync_copy(ew_v, ew_hbm.at[pl.ds(base * 16, ROWS * 16)])
    pltpu.sync_copy(ei_v, ei_hbm.at[pl.ds(base * 16, ROWS * 16)])
    pltpu.sync_copy(hist_v, hist_hbm.at[pl.ds(wid * N_EXP, N_EXP)])


_sc_topk = functools.partial(
    pl.kernel,
    out_type=[
        jax.ShapeDtypeStruct((N_TOK * 16,), jnp.float32),
        jax.ShapeDtypeStruct((N_TOK * 16,), jnp.int32),
        jax.ShapeDtypeStruct((NW * N_EXP,), jnp.int32),
    ],
    mesh=plsc.VectorSubcoreMesh(core_axis_name="c", subcore_axis_name="s"),
    scratch_types=[
        pltpu.VMEM((ROWS * N_EXP,), jnp.float32),
        pltpu.VMEM((ROWS * 16,), jnp.float32),
        pltpu.VMEM((ROWS * 16,), jnp.int32),
        pltpu.VMEM((N_EXP,), jnp.int32),
    ],
    compiler_params=pltpu.CompilerParams(needs_layout_passes=False),
)(_sc_body)


def kernel(x, W):
    logits, scores = _dense(x, W)
    ew16, ei16, hist = _sc_topk(scores.reshape(-1))
    ew = ew16.reshape(N_TOK, 16)[:, :TOPK]
    ei = ei16.reshape(N_TOK, 16)[:, :TOPK]
    return logits, scores, ew, ei, hist.reshape(NW, N_EXP).sum(0)


# speedup vs baseline: 1.0020x; 1.0020x over previous
import functools

import jax
import jax.numpy as jnp
from jax import lax
from jax.experimental import pallas as pl
from jax.experimental.pallas import tpu as pltpu
from jax.experimental.pallas import tpu_sc as plsc

D_MODEL = 2048
N_EXP = 64
TOPK = 8
N_TOK = 8192
BLK = 1024

_INFO = plsc.get_sparse_core_info()
_NC = _INFO.num_cores
_NS = _INFO.num_subcores
NW = _NC * _NS
ROWS = N_TOK // NW


def _dense_body(x_ref, w_ref, logits_ref, scores_ref):
    x = x_ref[...]
    w = w_ref[...]
    logits = jax.lax.dot_general(
        x, w, (((1,), (1,)), ((), ())), preferred_element_type=jnp.float32
    )
    logits_ref[...] = logits
    m = jnp.max(logits, axis=-1, keepdims=True)
    e = jnp.exp(logits - m)
    scores_ref[...] = e / jnp.sum(e, axis=-1, keepdims=True)


def _dense(x, W):
    return pl.pallas_call(
        _dense_body,
        grid=(N_TOK // BLK,),
        in_specs=[
            pl.BlockSpec((BLK, D_MODEL), lambda i: (i, 0)),
            pl.BlockSpec((N_EXP, D_MODEL), lambda i: (0, 0)),
        ],
        out_specs=[
            pl.BlockSpec((BLK, N_EXP), lambda i: (i, 0)),
            pl.BlockSpec((BLK, N_EXP), lambda i: (i, 0)),
        ],
        out_shape=[
            jax.ShapeDtypeStruct((N_TOK, N_EXP), jnp.float32),
            jax.ShapeDtypeStruct((N_TOK, N_EXP), jnp.float32),
        ],
    )(x, W)


def _sc_body(scores_hbm, ew_hbm, ei_hbm, hist_hbm, blk_v, ew_v, ei_v, hist_v):
    wid = lax.axis_index("s") * _NC + lax.axis_index("c")
    base = wid * ROWS

    pltpu.sync_copy(scores_hbm.at[pl.ds(base * N_EXP, ROWS * N_EXP)], blk_v)

    iota = lax.iota(jnp.int32, 16)
    lane_mask = iota < TOPK
    ones = jnp.ones((16,), jnp.int32)
    for j in range(4):
        hist_v[pl.ds(16 * j, 16)] = jnp.zeros((16,), jnp.int32)

    def merge(ka, va, kb, vb):
        kr = lax.rev(kb, (0,))
        vr = lax.rev(vb, (0,))
        sel = ka >= kr
        return plsc.sort_key_val(
            jnp.where(sel, ka, kr), jnp.where(sel, va, vr), descending=True
        )

    @plsc.parallel_loop(0, ROWS, 1, unroll=4)
    def _row_loop(r):
        off = r * N_EXP
        ks, vs = [], []
        for j in range(4):
            k, v = plsc.sort_key_val(
                blk_v[pl.ds(off + 16 * j, 16)], iota + 16 * j, descending=True
            )
            ks.append(k)
            vs.append(v)
        k01, v01 = merge(ks[0], vs[0], ks[1], vs[1])
        k23, v23 = merge(ks[2], vs[2], ks[3], vs[3])
        kf, vf = merge(k01, v01, k23, v23)
        ew_v[pl.ds(r * 16, 16)] = kf
        ei_v[pl.ds(r * 16, 16)] = vf
        plsc.addupdate_scatter(hist_v, [vf], ones, mask=lane_mask)

    pltpu.sync_copy(ew_v, ew_hbm.at[pl.ds(base * 16, ROWS * 16)])
    pltpu.sync_copy(ei_v, ei_hbm.at[pl.ds(base * 16, ROWS * 16)])
    pltpu.sync_copy(hist_v, hist_hbm.at[pl.ds(wid * N_EXP, N_EXP)])


_sc_topk = functools.partial(
    pl.kernel,
    out_type=[
        jax.ShapeDtypeStruct((N_TOK * 16,), jnp.float32),
        jax.ShapeDtypeStruct((N_TOK * 16,), jnp.int32),
        jax.ShapeDtypeStruct((NW * N_EXP,), jnp.int32),
    ],
    mesh=plsc.VectorSubcoreMesh(core_axis_name="c", subcore_axis_name="s"),
    scratch_types=[
        pltpu.VMEM((ROWS * N_EXP,), jnp.float32),
        pltpu.VMEM((ROWS * 16,), jnp.float32),
        pltpu.VMEM((ROWS * 16,), jnp.int32),
        pltpu.VMEM((N_EXP,), jnp.int32),
    ],
    compiler_params=pltpu.CompilerParams(needs_layout_passes=False),
)(_sc_body)


def kernel(x, W):
    logits, scores = _dense(x, W)
    ew16, ei16, hist = _sc_topk(scores.reshape(-1))
    ew = ew16.reshape(N_TOK, 16)[:, :TOPK]
    ei = ei16.reshape(N_TOK, 16)[:, :TOPK]
    return logits, scores, ew, ei, hist.reshape(NW, N_EXP).sum(0)
